# Initial kernel scaffold; baseline (speedup 1.0000x reference)
import functools

import jax
import jax.numpy as jnp
from jax import lax
from jax.experimental import pallas as pl
from jax.experimental.pallas import tpu as pltpu
from jax.experimental.pallas import tpu_sc as plsc

VOCAB = 100000
N_PREFIX = 200
EMBED_DIM = 64
BATCH = 4096
HIST = 200

NC = 2
NS = 16
NW = NC * NS

B_TOTAL = BATCH * HIST
B_PER_W = B_TOTAL // NW
GROUP = 128
G = B_PER_W // GROUP
NBUF = 4


def _sc_gather(table, idx2d):
    mesh = plsc.VectorSubcoreMesh(
        core_axis_name="c", subcore_axis_name="s", num_cores=NC, num_subcores=NS
    )

    @functools.partial(
        pl.kernel,
        out_type=jax.ShapeDtypeStruct((B_TOTAL, EMBED_DIM), jnp.float32),
        mesh=mesh,
        scratch_types=[
            pltpu.VMEM((G, GROUP), jnp.int32),
            pltpu.VMEM((NBUF, GROUP, EMBED_DIM), jnp.float32),
            pltpu.SemaphoreType.DMA,
            pltpu.SemaphoreType.DMA,
            pltpu.SemaphoreType.DMA,
            pltpu.SemaphoreType.DMA,
            pltpu.SemaphoreType.DMA,
            pltpu.SemaphoreType.DMA,
            pltpu.SemaphoreType.DMA,
            pltpu.SemaphoreType.DMA,
        ],
    )
    def body(table_hbm, idx_hbm, out_hbm, idx_v, rows, *sems):
        gsems = sems[:NBUF]
        ssems = sems[NBUF:]
        wid = lax.axis_index("s") * NC + lax.axis_index("c")
        gbase = wid * G

        pltpu.sync_copy(idx_hbm.at[pl.ds(gbase, G)], idx_v)

        def start_gather(g, b):
            pltpu.async_copy(table_hbm.at[idx_v.at[g]], rows.at[b], gsems[b])

        def wait_gather(b):
            pltpu.make_async_copy(
                table_hbm.at[idx_v.at[0]], rows.at[b], gsems[b]
            ).wait()

        def start_scatter(g, b):
            pltpu.async_copy(
                rows.at[b],
                out_hbm.at[pl.ds((gbase + g) * GROUP, GROUP)],
                ssems[b],
            )

        def wait_scatter(b):
            pltpu.make_async_copy(
                rows.at[b],
                out_hbm.at[pl.ds(gbase * GROUP, GROUP)],
                ssems[b],
            ).wait()

        start_gather(0, 0)
        start_gather(1, 1)

        def loop(j, carry):
            for b in range(NBUF):
                g = NBUF * j + b
                wait_gather(b)
                start_scatter(g, b)
                b2 = (b + 2) % NBUF
                if b < 2:
                    @pl.when(j > 0)
                    def _():
                        wait_scatter(b2)

                    start_gather(g + 2, b2)
                else:
                    @pl.when(j < G // NBUF - 1)
                    def _():
                        wait_scatter(b2)
                        start_gather(g + 2, b2)
            return carry

        lax.fori_loop(0, G // NBUF, loop, 0)

        for b in range(NBUF):
            wait_scatter(b)

    return body(table, idx2d)


@jax.jit
def kernel(input, W_embed, W_new):
    table = jnp.concatenate([W_embed, W_new], axis=0)
    idx = input.reshape(-1).astype(jnp.int32).reshape(NW * G, GROUP)
    out = _sc_gather(table, idx)
    return out.reshape(BATCH, HIST, EMBED_DIM)


# speedup vs baseline: 4.0901x; 4.0901x over previous
import functools

import jax
import jax.numpy as jnp
from jax import lax
from jax.experimental import pallas as pl
from jax.experimental.pallas import tpu as pltpu
from jax.experimental.pallas import tpu_sc as plsc

VOCAB = 100000
N_PREFIX = 200
EMBED_DIM = 64
BATCH = 4096
HIST = 200

NC = 2
NS = 16
NW = NC * NS

B_TOTAL = BATCH * HIST
B_PER_W = B_TOTAL // NW
GROUP = 128
G = B_PER_W // GROUP
NBUF = 4


def _sc_gather(table, idx2d):
    mesh = plsc.VectorSubcoreMesh(
        core_axis_name="c", subcore_axis_name="s", num_cores=NC, num_subcores=NS
    )

    @functools.partial(
        pl.kernel,
        out_type=jax.ShapeDtypeStruct((B_TOTAL, EMBED_DIM), jnp.float32),
        mesh=mesh,
        compiler_params=pltpu.CompilerParams(use_tc_tiling_on_sc=False),
        scratch_types=[
            pltpu.VMEM((G, GROUP), jnp.int32),
            pltpu.VMEM((NBUF, GROUP, EMBED_DIM), jnp.float32),
            pltpu.SemaphoreType.DMA,
            pltpu.SemaphoreType.DMA,
            pltpu.SemaphoreType.DMA,
            pltpu.SemaphoreType.DMA,
            pltpu.SemaphoreType.DMA,
            pltpu.SemaphoreType.DMA,
            pltpu.SemaphoreType.DMA,
            pltpu.SemaphoreType.DMA,
        ],
    )
    def body(table_hbm, idx_hbm, out_hbm, idx_v, rows, *sems):
        gsems = sems[:NBUF]
        ssems = sems[NBUF:]
        wid = lax.axis_index("s") * NC + lax.axis_index("c")
        gbase = wid * G

        pltpu.sync_copy(idx_hbm.at[pl.ds(gbase, G)], idx_v)

        def start_gather(g, b):
            pltpu.async_copy(table_hbm.at[idx_v.at[g]], rows.at[b], gsems[b])

        def wait_gather(b):
            pltpu.make_async_copy(
                table_hbm.at[idx_v.at[0]], rows.at[b], gsems[b]
            ).wait()

        def start_scatter(g, b):
            pltpu.async_copy(
                rows.at[b],
                out_hbm.at[pl.ds((gbase + g) * GROUP, GROUP)],
                ssems[b],
            )

        def wait_scatter(b):
            pltpu.make_async_copy(
                rows.at[b],
                out_hbm.at[pl.ds(gbase * GROUP, GROUP)],
                ssems[b],
            ).wait()

        start_gather(0, 0)
        start_gather(1, 1)

        def loop(j, carry):
            for b in range(NBUF):
                g = NBUF * j + b
                wait_gather(b)
                start_scatter(g, b)
                b2 = (b + 2) % NBUF
                if b < 2:
                    @pl.when(j > 0)
                    def _():
                        wait_scatter(b2)

                    start_gather(g + 2, b2)
                else:
                    @pl.when(j < G // NBUF - 1)
                    def _():
                        wait_scatter(b2)
                        start_gather(g + 2, b2)
            return carry

        lax.fori_loop(0, G // NBUF, loop, 0)

        for b in range(NBUF):
            wait_scatter(b)

    return body(table, idx2d)


@jax.jit
def kernel(input, W_embed, W_new):
    table = jnp.concatenate([W_embed, W_new], axis=0)
    idx = input.reshape(-1).astype(jnp.int32).reshape(NW * G, GROUP)
    out = _sc_gather(table, idx)
    return out.reshape(BATCH, HIST, EMBED_DIM)
